# baseline (speedup 1.0000x reference)
import jax
import jax.numpy as jnp
from jax import lax
from jax.experimental import pallas as pl
from jax.experimental.pallas import tpu as pltpu
from jax.experimental.pallas import tpu_sc as plsc

N_STYLE = 1000000
D_STYLE = 64
BATCH = 16384
HIST = 20

NC = 2
NS = 16
NW = NC * NS
LANES = 16

NIDX = BATCH * HIST
N_PER_W = NIDX // NW

NBLK = (N_STYLE + 127) // 128
BLK_PER_W = (NBLK + NW - 1) // NW
PAD_ROWS = NBLK * 128

CHUNK = 256
N_CHUNKS = N_PER_W // CHUNK

TAIL = (NBLK - 1) * 128


def _transpose_block(blk_v, t_v, width):
    def row_step(c, carry):
        col = jnp.broadcast_to(c, (LANES,)).astype(jnp.int32)
        for q in range(D_STYLE // LANES):
            d = lax.iota(jnp.int32, LANES) + q * LANES
            t_v[c, pl.ds(q * LANES, LANES)] = plsc.load_gather(blk_v, [d, col])
        return carry

    lax.fori_loop(0, width, row_step, 0, unroll=2)


def _transpose_body(lut_t_hbm, tail_hbm, scratch_hbm, blk_v, t_v):
    wid = lax.axis_index("s") * NC + lax.axis_index("c")

    def blk_step(jj, carry):
        j = jj * NW + wid

        @pl.when(j < NBLK - 1)
        def _full():
            c0 = j * 128
            pltpu.sync_copy(lut_t_hbm.at[:, pl.ds(c0, 128)], blk_v)
            _transpose_block(blk_v, t_v, 128)
            pltpu.sync_copy(t_v, scratch_hbm.at[pl.ds(c0, 128)])

        @pl.when(j == NBLK - 1)
        def _tail():
            pltpu.sync_copy(tail_hbm, t_v.at[pl.ds(0, D_STYLE)])
            pltpu.sync_copy(t_v.at[pl.ds(0, D_STYLE)],
                            scratch_hbm.at[pl.ds(TAIL, D_STYLE)])

        return carry

    lax.fori_loop(0, BLK_PER_W, blk_step, 0)


def _gather_body(idx_hbm, scratch_hbm, out_hbm, idx_v, rows_v, sem):
    wid = lax.axis_index("s") * NC + lax.axis_index("c")
    base = wid * N_PER_W
    pltpu.sync_copy(idx_hbm.at[pl.ds(base, N_PER_W)], idx_v)

    def chunk_step(k, carry):
        off = k * CHUNK
        idx_chunk = idx_v.at[pl.ds(off, CHUNK)]
        pltpu.async_copy(scratch_hbm.at[idx_chunk], rows_v, sem).wait()
        pltpu.sync_copy(rows_v, out_hbm.at[pl.ds(base + off, CHUNK)])
        return carry

    lax.fori_loop(0, N_CHUNKS, chunk_step, 0)


@jax.jit
def _embed(x_flat, lut_t, tail_pad):
    mesh = plsc.VectorSubcoreMesh(
        core_axis_name="c", subcore_axis_name="s", num_cores=NC,
        num_subcores=NS)

    transpose_k = pl.kernel(
        _transpose_body,
        out_type=jax.ShapeDtypeStruct((PAD_ROWS, 2 * D_STYLE), jnp.float32),
        mesh=mesh,
        scratch_types=[
            pltpu.VMEM((D_STYLE, 128), jnp.float32),
            pltpu.VMEM((128, 2 * D_STYLE), jnp.float32),
        ],
        compiler_params=pltpu.CompilerParams(needs_layout_passes=False),
    )
    scratch = transpose_k(lut_t, tail_pad)

    gather_k = pl.kernel(
        _gather_body,
        out_type=jax.ShapeDtypeStruct((NIDX, 2 * D_STYLE), jnp.float32),
        mesh=mesh,
        scratch_types=[
            pltpu.VMEM((N_PER_W,), jnp.int32),
            pltpu.VMEM((CHUNK, 2 * D_STYLE), jnp.float32),
            pltpu.SemaphoreType.DMA,
        ],
        compiler_params=pltpu.CompilerParams(needs_layout_passes=False),
    )
    out_pad = gather_k(x_flat, scratch)
    return out_pad[:, :D_STYLE]


def kernel(x, lut):
    x_flat = x.reshape(NIDX).astype(jnp.int32)
    tail_pad = jnp.pad(lut[TAIL:], ((0, 0), (0, 128 - D_STYLE)))
    out = _embed(x_flat, lut.T, tail_pad)
    return out.reshape(BATCH, HIST, D_STYLE)


# speedup vs baseline: 2.4835x; 2.4835x over previous
import jax
import jax.numpy as jnp
from jax import lax
from jax.experimental import pallas as pl
from jax.experimental.pallas import tpu as pltpu
from jax.experimental.pallas import tpu_sc as plsc

N_STYLE = 1000000
D_STYLE = 64
BATCH = 16384
HIST = 20

NC = 2
NS = 16
NW = NC * NS
LANES = 16

NIDX = BATCH * HIST
N_PER_W = NIDX // NW

CHUNK = 256
N_CHUNKS = N_PER_W // CHUNK


def _gather_body(idx_hbm, table_hbm, out_hbm, idx_v, rows_v, sem):
    wid = lax.axis_index("s") * NC + lax.axis_index("c")
    base = wid * N_PER_W
    pltpu.sync_copy(idx_hbm.at[pl.ds(base, N_PER_W)], idx_v)

    def chunk_step(k, carry):
        off = k * CHUNK
        idx_chunk = idx_v.at[pl.ds(off, CHUNK)]
        pltpu.async_copy(table_hbm.at[idx_chunk], rows_v, sem).wait()
        pltpu.sync_copy(rows_v, out_hbm.at[pl.ds(base + off, CHUNK)])
        return carry

    lax.fori_loop(0, N_CHUNKS, chunk_step, 0)


@jax.jit
def _embed(x_flat, lut):
    table = jnp.pad(lut, ((0, 0), (0, 128 - D_STYLE)))

    mesh = plsc.VectorSubcoreMesh(
        core_axis_name="c", subcore_axis_name="s", num_cores=NC,
        num_subcores=NS)
    gather_k = pl.kernel(
        _gather_body,
        out_type=jax.ShapeDtypeStruct((NIDX, 2 * D_STYLE), jnp.float32),
        mesh=mesh,
        scratch_types=[
            pltpu.VMEM((N_PER_W,), jnp.int32),
            pltpu.VMEM((CHUNK, 2 * D_STYLE), jnp.float32),
            pltpu.SemaphoreType.DMA,
        ],
        compiler_params=pltpu.CompilerParams(needs_layout_passes=False),
    )
    out_pad = gather_k(x_flat, table)
    return out_pad[:, :D_STYLE]


def kernel(x, lut):
    x_flat = x.reshape(NIDX).astype(jnp.int32)
    out = _embed(x_flat, lut)
    return out.reshape(BATCH, HIST, D_STYLE)
